# baseline (speedup 1.0000x reference)
import functools

import jax
import jax.numpy as jnp
from jax import lax
from jax.experimental import pallas as pl
from jax.experimental.pallas import tpu as pltpu
from jax.experimental.pallas import tpu_sc as plsc

_NUM_ROWS = 32
_NUM_COLS = 32
_EMBED_DIM = 768
_LANES = 16
_CHUNKS = _EMBED_DIM // _LANES
_RPT = 2
_SLAB = _NUM_COLS * _EMBED_DIM

_mesh = plsc.VectorSubcoreMesh(
    core_axis_name="c", subcore_axis_name="s", num_cores=1
)


@functools.partial(
    pl.kernel,
    mesh=_mesh,
    out_type=jax.ShapeDtypeStruct((_NUM_ROWS * _NUM_COLS * _EMBED_DIM,), jnp.float32),
    scratch_types=[
        pltpu.VMEM((_RPT * _EMBED_DIM,), jnp.float32),
        pltpu.VMEM((_RPT * _SLAB,), jnp.float32),
        pltpu.SemaphoreType.DMA,
        pltpu.SemaphoreType.DMA,
        pltpu.SemaphoreType.DMA,
    ],
)
def _pos2d(wrow_hbm, wcol_hbm, out_hbm, wr_v, out_v, rsem, csem, osem):
    wid = lax.axis_index("s")
    row_cp = pltpu.async_copy(
        wrow_hbm.at[pl.ds((1 + _RPT * wid) * _EMBED_DIM, _RPT * _EMBED_DIM)],
        wr_v,
        rsem,
    )
    init0 = pltpu.async_copy(
        wcol_hbm.at[pl.ds(_EMBED_DIM, _SLAB)], out_v.at[pl.ds(0, _SLAB)], csem
    )
    init1 = pltpu.async_copy(
        wcol_hbm.at[pl.ds(_EMBED_DIM, _SLAB)], out_v.at[pl.ds(_SLAB, _SLAB)], csem
    )
    row_cp.wait()
    init0.wait()
    init1.wait()

    for r in range(_RPT):
        wr_regs = [
            wr_v[pl.ds(r * _EMBED_DIM + j * _LANES, _LANES)] for j in range(_CHUNKS)
        ]

        def col_body(c, _):
            base = c * _EMBED_DIM
            for j in range(_CHUNKS):
                plsc.addupdate(out_v.at[pl.ds(base + j * _LANES, _LANES)], wr_regs[j])
            return 0

        lax.fori_loop(r * _NUM_COLS, (r + 1) * _NUM_COLS, col_body, 0)
        pltpu.async_copy(
            out_v.at[pl.ds(r * _SLAB, _SLAB)],
            out_hbm.at[pl.ds((_RPT * wid + r) * _SLAB, _SLAB)],
            osem,
        )

    pltpu.make_async_copy(
        out_v, out_hbm.at[pl.ds(_RPT * wid * _SLAB, _RPT * _SLAB)], osem
    ).wait()


def kernel(input, W_row, W_col):
    del input
    wr = W_row.reshape((1 + _NUM_ROWS) * _EMBED_DIM)
    wc = W_col.reshape((1 + _NUM_COLS) * _EMBED_DIM)
    out = _pos2d(wr, wc)
    return out.reshape(1, _NUM_ROWS * _NUM_COLS, _EMBED_DIM)


# speedup vs baseline: 1.2106x; 1.2106x over previous
import functools

import jax
import jax.numpy as jnp
from jax import lax
from jax.experimental import pallas as pl
from jax.experimental.pallas import tpu as pltpu
from jax.experimental.pallas import tpu_sc as plsc

_NUM_ROWS = 32
_NUM_COLS = 32
_EMBED_DIM = 768
_LANES = 16
_CHUNKS = _EMBED_DIM // _LANES
_RPT = 2

_mesh = plsc.VectorSubcoreMesh(
    core_axis_name="c", subcore_axis_name="s", num_cores=1
)


@functools.partial(
    pl.kernel,
    mesh=_mesh,
    out_type=jax.ShapeDtypeStruct((_NUM_ROWS * _NUM_COLS, _EMBED_DIM), jnp.float32),
    scratch_types=[
        pltpu.VMEM((_RPT * _EMBED_DIM,), jnp.float32),
        pltpu.VMEM((_RPT * _NUM_COLS, _EMBED_DIM), jnp.float32),
        pltpu.SemaphoreType.DMA,
        pltpu.SemaphoreType.DMA,
        pltpu.SemaphoreType.DMA,
    ],
)
def _pos2d(wrow_hbm, wcol_hbm, out_hbm, wr_v, out_v, rsem, csem, osem):
    wid = lax.axis_index("s")
    row_cp = pltpu.async_copy(
        wrow_hbm.at[pl.ds((1 + _RPT * wid) * _EMBED_DIM, _RPT * _EMBED_DIM)],
        wr_v,
        rsem,
    )
    init0 = pltpu.async_copy(wcol_hbm, out_v.at[pl.ds(0, _NUM_COLS)], csem)
    init1 = pltpu.async_copy(wcol_hbm, out_v.at[pl.ds(_NUM_COLS, _NUM_COLS)], csem)
    row_cp.wait()
    init0.wait()
    init1.wait()

    for r in range(_RPT):
        wr_regs = [
            wr_v[pl.ds(r * _EMBED_DIM + j * _LANES, _LANES)] for j in range(_CHUNKS)
        ]

        def col_body(c, _):
            for j in range(_CHUNKS):
                plsc.addupdate(out_v.at[c, pl.ds(j * _LANES, _LANES)], wr_regs[j])
            return 0

        lax.fori_loop(r * _NUM_COLS, (r + 1) * _NUM_COLS, col_body, 0)
        pltpu.async_copy(
            out_v.at[pl.ds(r * _NUM_COLS, _NUM_COLS)],
            out_hbm.at[pl.ds(wid * (_RPT * _NUM_COLS) + r * _NUM_COLS, _NUM_COLS)],
            osem,
        )

    pltpu.make_async_copy(
        out_v, out_hbm.at[pl.ds(wid * (_RPT * _NUM_COLS), _RPT * _NUM_COLS)], osem
    ).wait()


def kernel(input, W_row, W_col):
    del input
    wr = W_row.reshape((1 + _NUM_ROWS) * _EMBED_DIM)
    wc = W_col[1 : 1 + _NUM_COLS]
    out = _pos2d(wr, wc)
    return out.reshape(1, _NUM_ROWS * _NUM_COLS, _EMBED_DIM)
